# baseline (speedup 1.0000x reference)
import functools

import jax
import jax.numpy as jnp
from jax import lax
from jax.experimental import pallas as pl
from jax.experimental.pallas import tpu as pltpu
from jax.experimental.pallas import tpu_sc as plsc

_BATCH = 16


@functools.lru_cache(maxsize=None)
def _make_gather(B, W, n_groups):
    info = plsc.get_sparse_core_info()
    nc, ns = info.num_cores, info.num_subcores
    nw = nc * ns
    assert B % (8 * nw) == 0
    b_per_w = B // nw
    mesh = plsc.VectorSubcoreMesh(core_axis_name="c", subcore_axis_name="s")

    @functools.partial(
        pl.kernel,
        mesh=mesh,
        out_type=jax.ShapeDtypeStruct((B, W), jnp.float32),
        scratch_types=[
            pltpu.VMEM((b_per_w,), jnp.int32),
            pltpu.VMEM((b_per_w, W), jnp.float32),
            pltpu.SemaphoreType.DMA,
        ],
    )
    def gather_k(table_hbm, idx_hbm, out_hbm, idx_v, rows_v, sem):
        wid = lax.axis_index("s") * nc + lax.axis_index("c")
        base = wid * b_per_w
        pltpu.sync_copy(idx_hbm.at[pl.ds(base, b_per_w)], idx_v)

        def body(i, carry):
            v = idx_v[pl.ds(i * _BATCH, _BATCH)]
            for j in range(_BATCH):
                r = v[j]
                g = lax.shift_right_logical(r, 4)
                s = lax.bitwise_and(lax.shift_right_logical(r, 1), 7)
                pltpu.async_copy(
                    table_hbm.at[g, pl.ds(s, 1)],
                    rows_v.at[pl.ds(i * _BATCH + j, 1)],
                    sem,
                )
            return carry

        lax.fori_loop(0, b_per_w // _BATCH, body, 0)
        pltpu.make_async_copy(
            out_hbm.at[pl.ds(base, b_per_w)], rows_v, sem
        ).wait()
        pltpu.sync_copy(rows_v, out_hbm.at[pl.ds(base, b_per_w)])

    return gather_k


def _mm_add_t_body(c_ref, noise_t_ref, g_ref, par_ref, out_ref):
    d = c_ref.shape[0]
    w = g_ref.shape[1]
    eye_w = (
        lax.broadcasted_iota(jnp.int32, (w, w), 0)
        == lax.broadcasted_iota(jnp.int32, (w, w), 1)
    ).astype(jnp.float32)
    y = lax.dot_general(
        c_ref[...],
        noise_t_ref[...],
        (((1,), (0,)), ((), ())),
        preferred_element_type=jnp.float32,
    )
    h = lax.dot_general(
        eye_w,
        g_ref[...],
        (((1,), (1,)), ((), ())),
        preferred_element_type=jnp.float32,
    )
    h0 = h[:d, :]
    h1 = h[d:, :]
    m = par_ref[...]
    out_ref[...] = y + h0 + (h1 - h0) * m


def kernel(inds, noise, mns, c):
    B, D = noise.shape
    n = mns.shape[0]
    W = 2 * D
    inds = inds.astype(jnp.int32)
    table3 = mns.reshape(n // 16, 8, W)
    g = _make_gather(B, W, n // 16)(table3, inds)
    noise_t = noise.T
    par = (inds & 1).astype(jnp.float32).reshape(1, B)

    blk = 2048
    out_t = pl.pallas_call(
        _mm_add_t_body,
        grid=(B // blk,),
        in_specs=[
            pl.BlockSpec((D, D), lambda i: (0, 0)),
            pl.BlockSpec((D, blk), lambda i: (0, i)),
            pl.BlockSpec((blk, W), lambda i: (i, 0)),
            pl.BlockSpec((1, blk), lambda i: (0, i)),
        ],
        out_specs=pl.BlockSpec((D, blk), lambda i: (0, i)),
        out_shape=jax.ShapeDtypeStruct((D, B), jnp.float32),
    )(c, noise_t, g, par)
    return out_t.T


# speedup vs baseline: 2.5446x; 2.5446x over previous
import functools

import jax
import jax.numpy as jnp
from jax import lax
from jax.experimental import pallas as pl
from jax.experimental.pallas import tpu as pltpu
from jax.experimental.pallas import tpu_sc as plsc

_GRP = 8
_BATCH = 16


@functools.lru_cache(maxsize=None)
def _make_gather(B, D, n_groups):
    info = plsc.get_sparse_core_info()
    nc, ns = info.num_cores, info.num_subcores
    nw = nc * ns
    assert B % (8 * nw) == 0
    b_per_w = B // nw
    mesh = plsc.VectorSubcoreMesh(core_axis_name="c", subcore_axis_name="s")

    @functools.partial(
        pl.kernel,
        mesh=mesh,
        out_type=jax.ShapeDtypeStruct((B, D), jnp.float32),
        scratch_types=[
            pltpu.VMEM((b_per_w,), jnp.int32),
            pltpu.VMEM((b_per_w, D), jnp.float32),
            pltpu.SemaphoreType.DMA,
        ],
    )
    def gather_k(table_hbm, idx_hbm, out_hbm, idx_v, rows_v, sem):
        wid = lax.axis_index("s") * nc + lax.axis_index("c")
        base = wid * b_per_w
        pltpu.sync_copy(idx_hbm.at[pl.ds(base, b_per_w)], idx_v)

        def body(i, carry):
            v = idx_v[pl.ds(i * _BATCH, _BATCH)]
            for j in range(_BATCH):
                r = v[j]
                r_hi = lax.shift_right_logical(r, 3)
                r_lo = lax.bitwise_and(r, 7)
                pltpu.async_copy(
                    table_hbm.at[r_hi, pl.ds(r_lo, 1)],
                    rows_v.at[pl.ds(i * _BATCH + j, 1)],
                    sem,
                )
            return carry

        lax.fori_loop(0, b_per_w // _BATCH, body, 0)
        pltpu.make_async_copy(
            out_hbm.at[pl.ds(base, b_per_w)], rows_v, sem
        ).wait()
        pltpu.sync_copy(rows_v, out_hbm.at[pl.ds(base, b_per_w)])

    return gather_k


def _mm_t_body(c_ref, noise_t_ref, y_ref):
    y_ref[...] = lax.dot_general(
        c_ref[...],
        noise_t_ref[...],
        (((1,), (0,)), ((), ())),
        preferred_element_type=jnp.float32,
    )


def _add_t_body(y_ref, g_ref, out_ref):
    d = y_ref.shape[0]
    eye = (
        lax.broadcasted_iota(jnp.int32, (d, d), 0)
        == lax.broadcasted_iota(jnp.int32, (d, d), 1)
    ).astype(jnp.float32)
    g_t = lax.dot_general(
        eye,
        g_ref[...],
        (((1,), (1,)), ((), ())),
        preferred_element_type=jnp.float32,
    )
    out_ref[...] = y_ref[...] + g_t


def kernel(inds, noise, mns, c):
    B, D = noise.shape
    n = mns.shape[0]
    table3 = mns.reshape(n // _GRP, _GRP, D)
    g = _make_gather(B, D, n // _GRP)(table3, inds.astype(jnp.int32))
    noise_t = noise.T

    blk = 4096
    y_t = pl.pallas_call(
        _mm_t_body,
        grid=(B // blk,),
        in_specs=[
            pl.BlockSpec((D, D), lambda i: (0, 0)),
            pl.BlockSpec((D, blk), lambda i: (0, i)),
        ],
        out_specs=pl.BlockSpec((D, blk), lambda i: (0, i)),
        out_shape=jax.ShapeDtypeStruct((D, B), jnp.float32),
    )(c, noise_t)
    out_t = pl.pallas_call(
        _add_t_body,
        grid=(B // blk,),
        in_specs=[
            pl.BlockSpec((D, blk), lambda i: (0, i)),
            pl.BlockSpec((blk, D), lambda i: (i, 0)),
        ],
        out_specs=pl.BlockSpec((D, blk), lambda i: (0, i)),
        out_shape=jax.ShapeDtypeStruct((D, B), jnp.float32),
    )(y_t, g)
    return out_t.T


# speedup vs baseline: 2.5534x; 1.0035x over previous
import functools

import jax
import jax.numpy as jnp
from jax import lax
from jax.experimental import pallas as pl
from jax.experimental.pallas import tpu as pltpu
from jax.experimental.pallas import tpu_sc as plsc

_GRP = 8
_BATCH = 16


@functools.lru_cache(maxsize=None)
def _make_gather(B, D, n_groups):
    info = plsc.get_sparse_core_info()
    nc, ns = info.num_cores, info.num_subcores
    nw = nc * ns
    assert B % (8 * nw) == 0
    b_per_w = B // nw
    mesh = plsc.VectorSubcoreMesh(core_axis_name="c", subcore_axis_name="s")

    @functools.partial(
        pl.kernel,
        mesh=mesh,
        out_type=jax.ShapeDtypeStruct((B, D), jnp.float32),
        scratch_types=[
            pltpu.VMEM((b_per_w,), jnp.int32),
            pltpu.VMEM((b_per_w, D), jnp.float32),
            pltpu.SemaphoreType.DMA,
        ],
    )
    def gather_k(table_hbm, idx_hbm, out_hbm, idx_v, rows_v, sem):
        wid = lax.axis_index("s") * nc + lax.axis_index("c")
        base = wid * b_per_w
        pltpu.sync_copy(idx_hbm.at[pl.ds(base, b_per_w)], idx_v)

        def body(i, carry):
            v = idx_v[pl.ds(i * _BATCH, _BATCH)]
            for j in range(_BATCH):
                r = v[j]
                r_hi = lax.shift_right_logical(r, 3)
                r_lo = lax.bitwise_and(r, 7)
                pltpu.async_copy(
                    table_hbm.at[r_hi, pl.ds(r_lo, 1)],
                    rows_v.at[pl.ds(i * _BATCH + j, 1)],
                    sem,
                )
            return carry

        lax.fori_loop(0, b_per_w // _BATCH, body, 0)
        pltpu.make_async_copy(
            out_hbm.at[pl.ds(base, b_per_w)], rows_v, sem
        ).wait()
        pltpu.sync_copy(rows_v, out_hbm.at[pl.ds(base, b_per_w)])

    return gather_k


def _mm_t_body(c_ref, noise_t_ref, y_ref):
    y_ref[...] = lax.dot_general(
        c_ref[...],
        noise_t_ref[...],
        (((1,), (0,)), ((), ())),
        preferred_element_type=jnp.float32,
    )


def _add_t_body(y_ref, g_ref, out_ref):
    d = y_ref.shape[0]
    eye = (
        lax.broadcasted_iota(jnp.int32, (d, d), 0)
        == lax.broadcasted_iota(jnp.int32, (d, d), 1)
    ).astype(jnp.float32)
    g_t = lax.dot_general(
        eye,
        g_ref[...],
        (((1,), (1,)), ((), ())),
        preferred_element_type=jnp.float32,
    )
    out_ref[...] = y_ref[...] + g_t


def kernel(inds, noise, mns, c):
    B, D = noise.shape
    n = mns.shape[0]
    table3 = mns.reshape(n // _GRP, _GRP, D)
    g = _make_gather(B, D, n // _GRP)(table3, inds.astype(jnp.int32))
    noise_t = noise.T

    blk = 4096
    y_t = pl.pallas_call(
        _mm_t_body,
        grid=(B // blk,),
        in_specs=[
            pl.BlockSpec((D, D), lambda i: (0, 0)),
            pl.BlockSpec((D, blk), lambda i: (0, i)),
        ],
        out_specs=pl.BlockSpec((D, blk), lambda i: (0, i)),
        out_shape=jax.ShapeDtypeStruct((D, B), jnp.float32),
    )(c, noise_t)
    blk2 = 8192
    out_t = pl.pallas_call(
        _add_t_body,
        grid=(B // blk2,),
        in_specs=[
            pl.BlockSpec((D, blk2), lambda i: (0, i)),
            pl.BlockSpec((blk2, D), lambda i: (i, 0)),
        ],
        out_specs=pl.BlockSpec((D, blk2), lambda i: (0, i)),
        out_shape=jax.ShapeDtypeStruct((D, B), jnp.float32),
    )(y_t, g)
    return out_t.T
